# baseline (speedup 1.0000x reference)
import functools

import jax
import jax.numpy as jnp
from jax import lax
from jax.experimental import pallas as pl
from jax.experimental.pallas import tpu as pltpu
from jax.experimental.pallas import tpu_sc as plsc

B = 8
N = 8192
QT = 2048
NQ = N // QT
NSC = 32
KQ = N // 4
BIG = 1 << 30


def _tc_body(q2_ref, kt_ref, tz_ref, pz_ref, idx_ref, ld_ref, s_ref):
    kx = kt_ref[0, 0:1, :]
    ky = kt_ref[0, 1:2, :]
    kk = 0.25 * (kx * kx + ky * ky)
    q2 = q2_ref[0]
    qk = lax.dot_general(q2, kt_ref[0], (((1,), (0,)), ((), ())),
                         preferred_element_type=jnp.float32)
    qx = q2[:, 0:1]
    qy = q2[:, 1:2]
    qq = qx * qx + qy * qy
    d2 = (qq + kk) + qk
    amin = jnp.argmin(d2, axis=1).astype(jnp.int32).reshape(QT, 1)
    tz = tz_ref[0]
    out_idx = jnp.where(tz != 0.0, amin, BIG)
    idx_ref[...] = out_idx.reshape(1, 1, QT, 1)

    pz = pz_ref[0]
    logp = jnp.maximum(jnp.log(pz), -100.0)
    log1mp = jnp.maximum(jnp.log(1.0 - pz), -100.0)
    ld_ref[...] = (log1mp - logp).reshape(1, 1, QT, 1)
    s_ref[...] = jnp.sum(-log1mp).reshape(1, 1, 1, 1)


def _tc_nn(q4, kt, tz, pz):
    return pl.pallas_call(
        _tc_body,
        grid=(B, NQ),
        in_specs=[
            pl.BlockSpec((1, QT, 2), lambda b, t: (b, t, 0)),
            pl.BlockSpec((1, 2, N), lambda b, t: (b, 0, 0)),
            pl.BlockSpec((1, QT, 1), lambda b, t: (b, t, 0)),
            pl.BlockSpec((1, QT, 1), lambda b, t: (b, t, 0)),
        ],
        out_specs=[
            pl.BlockSpec((1, 1, QT, 1), lambda b, t: (b, t, 0, 0)),
            pl.BlockSpec((1, 1, QT, 1), lambda b, t: (b, t, 0, 0)),
            pl.BlockSpec((1, 1, 1, 1), lambda b, t: (b, t, 0, 0)),
        ],
        out_shape=[
            jax.ShapeDtypeStruct((B, NQ, QT, 1), jnp.int32),
            jax.ShapeDtypeStruct((B, NQ, QT, 1), jnp.float32),
            jax.ShapeDtypeStruct((B, NQ, 1, 1), jnp.float32),
        ],
        compiler_params=pltpu.CompilerParams(
            dimension_semantics=("parallel", "parallel")),
    )(q4, kt, tz, pz)


def _sc_body(idx_hbm, ld_hbm, out_hbm, idx_v, ld_v, marked_v, part_v):
    c = lax.axis_index("c")
    s = lax.axis_index("s")
    wid = s * 2 + c
    b = wid // 4
    q = wid % 4
    base_q = b * N
    base_k = q * KQ

    pltpu.sync_copy(idx_hbm.at[pl.ds(base_q, N)], idx_v)
    pltpu.sync_copy(ld_hbm.at[pl.ds(base_q + base_k, KQ)], ld_v)

    zeros16 = jnp.zeros((16,), jnp.float32)
    ones16 = jnp.ones((16,), jnp.float32)

    @pl.loop(0, KQ // 16)
    def _zero(i):
        marked_v[pl.ds(i * 16, 16)] = zeros16

    @pl.loop(0, N // 16)
    def _scatter(j):
        v = idx_v[pl.ds(j * 16, 16)]
        rel = v - base_k
        inb = (rel >= 0) & (rel < KQ)
        relc = jnp.clip(rel, 0, KQ - 1)
        plsc.store_scatter(marked_v, [relc], ones16, mask=inb)

    def _red(i, acc):
        return acc + marked_v[pl.ds(i * 16, 16)] * ld_v[pl.ds(i * 16, 16)]

    acc = lax.fori_loop(0, KQ // 16, _red, jnp.zeros((16,), jnp.float32))
    part_v[...] = acc
    pltpu.sync_copy(part_v, out_hbm.at[wid])


def _sc_scatter_reduce(idx_flat, ld_flat):
    mesh = plsc.VectorSubcoreMesh(core_axis_name="c", subcore_axis_name="s")
    fn = functools.partial(
        pl.kernel,
        out_type=jax.ShapeDtypeStruct((NSC, 16), jnp.float32),
        mesh=mesh,
        compiler_params=pltpu.CompilerParams(needs_layout_passes=False),
        scratch_types=[
            pltpu.VMEM((N,), jnp.int32),
            pltpu.VMEM((KQ,), jnp.float32),
            pltpu.VMEM((KQ,), jnp.float32),
            pltpu.VMEM((16,), jnp.float32),
        ],
    )(_sc_body)
    return fn(idx_flat, ld_flat)


def kernel(preds, targs, label_lengths):
    del label_lengths
    q2 = targs[:, :, 0:2]
    kt = jnp.transpose(preds[:, :, 0:2] * -2.0, (0, 2, 1))
    tz = targs[:, :, 2:3]
    pz = preds[:, :, 2:3]

    idx4, ld4, s4 = _tc_nn(q2, kt, tz, pz)
    idx_flat = idx4.reshape(B * N)
    ld_flat = ld4.reshape(B * N)

    parts = _sc_scatter_reduce(idx_flat, ld_flat)
    loss = (jnp.sum(s4) + jnp.sum(parts)) / jnp.float32(N)
    return loss.astype(jnp.float32)


# speedup vs baseline: 1.0204x; 1.0204x over previous
import functools

import jax
import jax.numpy as jnp
from jax import lax
from jax.experimental import pallas as pl
from jax.experimental.pallas import tpu as pltpu
from jax.experimental.pallas import tpu_sc as plsc

B = 8
N = 8192
QT = 1024
NQ = N // QT
NSC = 32
KQ = N // 4
BIG = 1 << 30


def _tc_body(q2_ref, kt_ref, tz_ref, pz_ref, idx_ref, ld_ref, s_ref):
    kx = kt_ref[0, 0:1, :]
    ky = kt_ref[0, 1:2, :]
    kk = 0.25 * (kx * kx + ky * ky)
    q2 = q2_ref[0]
    qk = lax.dot_general(q2, kt_ref[0], (((1,), (0,)), ((), ())),
                         preferred_element_type=jnp.float32)
    qx = q2[:, 0:1]
    qy = q2[:, 1:2]
    qq = qx * qx + qy * qy
    d2 = (qq + kk) + qk
    amin = jnp.argmin(d2, axis=1).astype(jnp.int32).reshape(QT, 1)
    tz = tz_ref[0]
    out_idx = jnp.where(tz != 0.0, amin, BIG)
    idx_ref[...] = out_idx.reshape(1, 1, QT, 1)

    pz = pz_ref[0]
    logp = jnp.maximum(jnp.log(pz), -100.0)
    log1mp = jnp.maximum(jnp.log(1.0 - pz), -100.0)
    ld_ref[...] = (log1mp - logp).reshape(1, 1, QT, 1)
    s_ref[...] = jnp.sum(-log1mp).reshape(1, 1, 1, 1)


def _tc_nn(q4, kt, tz, pz):
    return pl.pallas_call(
        _tc_body,
        grid=(B, NQ),
        in_specs=[
            pl.BlockSpec((1, QT, 2), lambda b, t: (b, t, 0)),
            pl.BlockSpec((1, 2, N), lambda b, t: (b, 0, 0)),
            pl.BlockSpec((1, QT, 1), lambda b, t: (b, t, 0)),
            pl.BlockSpec((1, QT, 1), lambda b, t: (b, t, 0)),
        ],
        out_specs=[
            pl.BlockSpec((1, 1, QT, 1), lambda b, t: (b, t, 0, 0)),
            pl.BlockSpec((1, 1, QT, 1), lambda b, t: (b, t, 0, 0)),
            pl.BlockSpec((1, 1, 1, 1), lambda b, t: (b, t, 0, 0)),
        ],
        out_shape=[
            jax.ShapeDtypeStruct((B, NQ, QT, 1), jnp.int32),
            jax.ShapeDtypeStruct((B, NQ, QT, 1), jnp.float32),
            jax.ShapeDtypeStruct((B, NQ, 1, 1), jnp.float32),
        ],
        compiler_params=pltpu.CompilerParams(
            dimension_semantics=("parallel", "parallel")),
    )(q4, kt, tz, pz)


def _sc_body(idx_hbm, ld_hbm, out_hbm, idx_v, ld_v, marked_v, part_v):
    c = lax.axis_index("c")
    s = lax.axis_index("s")
    wid = s * 2 + c
    b = wid // 4
    q = wid % 4
    base_q = b * N
    base_k = q * KQ

    pltpu.sync_copy(idx_hbm.at[pl.ds(base_q, N)], idx_v)
    pltpu.sync_copy(ld_hbm.at[pl.ds(base_q + base_k, KQ)], ld_v)

    zeros16 = jnp.zeros((16,), jnp.float32)
    ones16 = jnp.ones((16,), jnp.float32)

    @pl.loop(0, KQ // 16)
    def _zero(i):
        marked_v[pl.ds(i * 16, 16)] = zeros16

    @pl.loop(0, N // 16)
    def _scatter(j):
        v = idx_v[pl.ds(j * 16, 16)]
        rel = v - base_k
        inb = (rel >= 0) & (rel < KQ)
        relc = jnp.clip(rel, 0, KQ - 1)
        plsc.store_scatter(marked_v, [relc], ones16, mask=inb)

    def _red(i, acc):
        return acc + marked_v[pl.ds(i * 16, 16)] * ld_v[pl.ds(i * 16, 16)]

    acc = lax.fori_loop(0, KQ // 16, _red, jnp.zeros((16,), jnp.float32))
    part_v[...] = acc
    pltpu.sync_copy(part_v, out_hbm.at[wid])


def _sc_scatter_reduce(idx_flat, ld_flat):
    mesh = plsc.VectorSubcoreMesh(core_axis_name="c", subcore_axis_name="s")
    fn = functools.partial(
        pl.kernel,
        out_type=jax.ShapeDtypeStruct((NSC, 16), jnp.float32),
        mesh=mesh,
        compiler_params=pltpu.CompilerParams(needs_layout_passes=False),
        scratch_types=[
            pltpu.VMEM((N,), jnp.int32),
            pltpu.VMEM((KQ,), jnp.float32),
            pltpu.VMEM((KQ,), jnp.float32),
            pltpu.VMEM((16,), jnp.float32),
        ],
    )(_sc_body)
    return fn(idx_flat, ld_flat)


def kernel(preds, targs, label_lengths):
    del label_lengths
    q2 = targs[:, :, 0:2]
    kt = jnp.transpose(preds[:, :, 0:2] * -2.0, (0, 2, 1))
    tz = targs[:, :, 2:3]
    pz = preds[:, :, 2:3]

    idx4, ld4, s4 = _tc_nn(q2, kt, tz, pz)
    idx_flat = idx4.reshape(B * N)
    ld_flat = ld4.reshape(B * N)

    parts = _sc_scatter_reduce(idx_flat, ld_flat)
    loss = (jnp.sum(s4) + jnp.sum(parts)) / jnp.float32(N)
    return loss.astype(jnp.float32)
